# baseline (speedup 1.0000x reference)
import functools

import jax
import jax.numpy as jnp
from jax import lax
from jax.experimental import pallas as pl
from jax.experimental.pallas import tpu as pltpu
from jax.experimental.pallas import tpu_sc as plsc

P, Q, V, D = 4096, 128, 100000, 300
POS_V, POS_D, NER_V, NER_D = 50, 12, 20, 8
SIM_D, H1, H2 = 280, 512, 256

NC, NS = 2, 16
NW = NC * NS
PB_SC = P // NW
QB_SC = 8

BP = 512


D_MAIN = 256
D_TAIL = D - D_MAIN
D_TAILP = 128

BLKE = 2000


def _tail_extract(word_table):
    def body(in_ref, out_ref):
        out_ref[...] = in_ref[...]

    return pl.pallas_call(
        body,
        grid=(V // BLKE,),
        in_specs=[pl.BlockSpec((BLKE, 128), lambda i: (i, 2))],
        out_specs=pl.BlockSpec((BLKE, D_TAILP), lambda i: (i, 0)),
        out_shape=jax.ShapeDtypeStruct((V, D_TAILP), jnp.float32),
    )(word_table)


DWIDE = 384


def _sc_gather_wide(paragraph_ids, question_ids, table_wide):
    mesh = plsc.VectorSubcoreMesh(core_axis_name="c", subcore_axis_name="s")

    @functools.partial(
        pl.kernel,
        mesh=mesh,
        compiler_params=pltpu.CompilerParams(use_tc_tiling_on_sc=True),
        out_type=[
            jax.ShapeDtypeStruct((P, DWIDE), jnp.float32),
            jax.ShapeDtypeStruct((Q, DWIDE), jnp.float32),
        ],
        scratch_types=[
            pltpu.VMEM((PB_SC,), jnp.int32),
            pltpu.VMEM((PB_SC, DWIDE), jnp.float32),
            pltpu.VMEM((QB_SC,), jnp.int32),
            pltpu.VMEM((QB_SC, DWIDE), jnp.float32),
            pltpu.SemaphoreType.DMA,
            pltpu.SemaphoreType.DMA,
        ],
    )
    def gather_kernel(pids_hbm, qids_hbm, table_hbm, p_out, q_out,
                      pidx_v, prow_v, qidx_v, qrow_v, psem, qsem):
        wid = lax.axis_index("s") * NC + lax.axis_index("c")
        base = wid * PB_SC
        pltpu.sync_copy(pids_hbm.at[pl.ds(base, PB_SC)], pidx_v)
        pcopy = pltpu.async_copy(table_hbm.at[pidx_v], prow_v, psem)

        @pl.when(wid < Q // QB_SC)
        def _():
            qbase = wid * QB_SC
            pltpu.sync_copy(qids_hbm.at[pl.ds(qbase, QB_SC)], qidx_v)
            pltpu.async_copy(table_hbm.at[qidx_v], qrow_v, qsem).wait()
            pltpu.sync_copy(qrow_v, q_out.at[pl.ds(qbase, QB_SC)])

        pcopy.wait()
        pltpu.sync_copy(prow_v, p_out.at[pl.ds(base, PB_SC)])

    return gather_kernel(paragraph_ids, question_ids, table_wide)


def _tc_body(p_ref, q_ref, pos_ref, ner_ref, em_ref, post_ref, nert_ref,
             gw_ref, gb_ref, qw1_ref, qb1_ref, qw2_ref, qb2_ref,
             w_word_ref, w_pos_ref, w_ner_ref, w_em_ref, w_sim_ref, b1_ref,
             w2_ref, b2_ref, pout_ref, qout_ref):
    i = pl.program_id(0)
    q = q_ref[...]

    @pl.when(i == 0)
    def _():
        hq = jnp.maximum(
            jnp.dot(q, qw1_ref[...], preferred_element_type=jnp.float32)
            + qb1_ref[...], 0.0)
        qout_ref[...] = jnp.maximum(
            jnp.dot(hq, qw2_ref[...], preferred_element_type=jnp.float32)
            + qb2_ref[...], 0.0)

    p = p_ref[...]
    qn = q / (jnp.sqrt(jnp.sum(q * q, axis=1, keepdims=True)) + 1e-8)
    s = jnp.sum(qn, axis=0, keepdims=True)
    pnorm = jnp.sqrt(jnp.sum(p * p, axis=1, keepdims=True))
    ws = jnp.sum(p * s, axis=1, keepdims=True) / (pnorm + 1e-8)
    sim = jnp.maximum(
        jnp.dot(p, gw_ref[...], preferred_element_type=jnp.float32)
        + gb_ref[...], 0.0) * ws
    pos_oh = (pos_ref[...] == lax.broadcasted_iota(jnp.int32, (BP, POS_V), 1)
              ).astype(jnp.float32)
    ner_oh = (ner_ref[...] == lax.broadcasted_iota(jnp.int32, (BP, NER_V), 1)
              ).astype(jnp.float32)
    pos_emb = jnp.dot(pos_oh, post_ref[...], preferred_element_type=jnp.float32)
    ner_emb = jnp.dot(ner_oh, nert_ref[...], preferred_element_type=jnp.float32)
    h = (jnp.dot(p, w_word_ref[...], preferred_element_type=jnp.float32)
         + jnp.dot(sim, w_sim_ref[...], preferred_element_type=jnp.float32)
         + jnp.dot(pos_emb, w_pos_ref[...], preferred_element_type=jnp.float32)
         + jnp.dot(ner_emb, w_ner_ref[...], preferred_element_type=jnp.float32)
         + em_ref[...] * w_em_ref[...]
         + b1_ref[...])
    h = jnp.maximum(h, 0.0)
    pout_ref[...] = jnp.maximum(
        jnp.dot(h, w2_ref[...], preferred_element_type=jnp.float32)
        + b2_ref[...], 0.0)


def _tc_compute(p_emb, q_emb, pos2, ner2, exact_match, pos_table, ner_table,
                g_w, g_b, q_w1, q_b1, q_w2, q_b2,
                w_word, w_pos, w_ner, w_em, w_sim, b1, w2, b2,
                interpret=False):
    full = lambda shape: pl.BlockSpec(shape, lambda i: (0, 0))
    blk = lambda shape: pl.BlockSpec(shape, lambda i: (i, 0))
    return pl.pallas_call(
        _tc_body,
        grid=(P // BP,),
        in_specs=[
            blk((BP, D)),
            full((Q, D)),
            blk((BP, 1)),
            blk((BP, 1)),
            blk((BP, 1)),
            full((POS_V, POS_D)),
            full((NER_V, NER_D)),
            full((D, SIM_D)),
            full((1, SIM_D)),
            full((D, H1)),
            full((1, H1)),
            full((H1, H2)),
            full((1, H2)),
            full((D, H1)),
            full((POS_D, H1)),
            full((NER_D, H1)),
            full((1, H1)),
            full((SIM_D, H1)),
            full((1, H1)),
            full((H1, H2)),
            full((1, H2)),
        ],
        out_specs=[
            blk((BP, H2)),
            full((Q, H2)),
        ],
        out_shape=[
            jax.ShapeDtypeStruct((P, H2), jnp.float32),
            jax.ShapeDtypeStruct((Q, H2), jnp.float32),
        ],
        interpret=interpret,
    )(p_emb, q_emb, pos2, ner2, exact_match, pos_table, ner_table,
      g_w, g_b, q_w1, q_b1, q_w2, q_b2,
      w_word, w_pos, w_ner, w_em, w_sim, b1, w2, b2)


def kernel(paragraph_ids, paragraph_pos, paragraph_ner, exact_match,
           question_ids, word_table, pos_table, ner_table, g_w, g_b,
           q_w1, q_b1, q_w2, q_b2, d_w1, d_b1, d_w2, d_b2):
    table_wide = jnp.pad(word_table, ((0, 0), (0, DWIDE - D)))
    p_wide, q_wide = _sc_gather_wide(paragraph_ids, question_ids, table_wide)
    p_emb = p_wide[:, :D]
    q_emb = q_wide[:, :D]

    pos2 = paragraph_pos.reshape(P, 1)
    ner2 = paragraph_ner.reshape(P, 1)
    w_word = d_w1[0:D]
    w_pos = d_w1[D:D + POS_D]
    w_ner = d_w1[D + POS_D:D + POS_D + NER_D]
    w_em = d_w1[D + POS_D + NER_D:D + POS_D + NER_D + 1]
    w_sim = d_w1[D + POS_D + NER_D + 1:]

    pout, qout = _tc_compute(
        p_emb, q_emb, pos2, ner2, exact_match, pos_table, ner_table,
        g_w, g_b.reshape(1, SIM_D), q_w1, q_b1.reshape(1, H1),
        q_w2, q_b2.reshape(1, H2),
        w_word, w_pos, w_ner, w_em, w_sim,
        d_b1.reshape(1, H1), d_w2, d_b2.reshape(1, H2))

    return (pout[None], qout[None], p_emb, q_emb)


# speedup vs baseline: 4.2392x; 4.2392x over previous
import functools

import jax
import jax.numpy as jnp
from jax import lax
from jax.experimental import pallas as pl
from jax.experimental.pallas import tpu as pltpu
from jax.experimental.pallas import tpu_sc as plsc

P, Q, V, D = 4096, 128, 100000, 300
POS_V, POS_D, NER_V, NER_D = 50, 12, 20, 8
SIM_D, H1, H2 = 280, 512, 256

NC, NS = 2, 16
NW = NC * NS
PB_SC = P // NW
QB_SC = 8

BP = 512


D_MAIN = 256
D_TAIL = D - D_MAIN
D_TAILP = 128

BLKE = 2048


def _transpose_pad(word_table_t):
    def body(in_ref, out_ref):
        out_ref[:, :D] = lax.transpose(in_ref[...], (1, 0))

    return pl.pallas_call(
        body,
        grid=(pl.cdiv(V, BLKE),),
        in_specs=[pl.BlockSpec((D, BLKE), lambda i: (0, i))],
        out_specs=pl.BlockSpec((BLKE, DWIDE), lambda i: (i, 0)),
        out_shape=jax.ShapeDtypeStruct((V, DWIDE), jnp.float32),
    )(word_table_t)


DWIDE = 384


def _sc_gather_wide(paragraph_ids, question_ids, table_wide):
    mesh = plsc.VectorSubcoreMesh(core_axis_name="c", subcore_axis_name="s")

    @functools.partial(
        pl.kernel,
        mesh=mesh,
        compiler_params=pltpu.CompilerParams(use_tc_tiling_on_sc=True),
        out_type=[
            jax.ShapeDtypeStruct((P, DWIDE), jnp.float32),
            jax.ShapeDtypeStruct((Q, DWIDE), jnp.float32),
        ],
        scratch_types=[
            pltpu.VMEM((PB_SC,), jnp.int32),
            pltpu.VMEM((PB_SC, DWIDE), jnp.float32),
            pltpu.VMEM((QB_SC,), jnp.int32),
            pltpu.VMEM((QB_SC, DWIDE), jnp.float32),
            pltpu.SemaphoreType.DMA,
            pltpu.SemaphoreType.DMA,
        ],
    )
    def gather_kernel(pids_hbm, qids_hbm, table_hbm, p_out, q_out,
                      pidx_v, prow_v, qidx_v, qrow_v, psem, qsem):
        wid = lax.axis_index("s") * NC + lax.axis_index("c")
        base = wid * PB_SC
        pltpu.sync_copy(pids_hbm.at[pl.ds(base, PB_SC)], pidx_v)
        pcopy = pltpu.async_copy(table_hbm.at[pidx_v], prow_v, psem)

        @pl.when(wid < Q // QB_SC)
        def _():
            qbase = wid * QB_SC
            pltpu.sync_copy(qids_hbm.at[pl.ds(qbase, QB_SC)], qidx_v)
            pltpu.async_copy(table_hbm.at[qidx_v], qrow_v, qsem).wait()
            pltpu.sync_copy(qrow_v, q_out.at[pl.ds(qbase, QB_SC)])

        pcopy.wait()
        pltpu.sync_copy(prow_v, p_out.at[pl.ds(base, PB_SC)])

    return gather_kernel(paragraph_ids, question_ids, table_wide)


def _tc_body(p_ref, q_ref, pos_ref, ner_ref, em_ref, post_ref, nert_ref,
             gw_ref, gb_ref, qw1_ref, qb1_ref, qw2_ref, qb2_ref,
             w_word_ref, w_pos_ref, w_ner_ref, w_em_ref, w_sim_ref, b1_ref,
             w2_ref, b2_ref, pout_ref, qout_ref):
    i = pl.program_id(0)
    q = q_ref[...]

    @pl.when(i == 0)
    def _():
        hq = jnp.maximum(
            jnp.dot(q, qw1_ref[...], preferred_element_type=jnp.float32)
            + qb1_ref[...], 0.0)
        qout_ref[...] = jnp.maximum(
            jnp.dot(hq, qw2_ref[...], preferred_element_type=jnp.float32)
            + qb2_ref[...], 0.0)

    p = p_ref[...]
    qn = q / (jnp.sqrt(jnp.sum(q * q, axis=1, keepdims=True)) + 1e-8)
    s = jnp.sum(qn, axis=0, keepdims=True)
    pnorm = jnp.sqrt(jnp.sum(p * p, axis=1, keepdims=True))
    ws = jnp.sum(p * s, axis=1, keepdims=True) / (pnorm + 1e-8)
    sim = jnp.maximum(
        jnp.dot(p, gw_ref[...], preferred_element_type=jnp.float32)
        + gb_ref[...], 0.0) * ws
    pos_oh = (pos_ref[...] == lax.broadcasted_iota(jnp.int32, (BP, POS_V), 1)
              ).astype(jnp.float32)
    ner_oh = (ner_ref[...] == lax.broadcasted_iota(jnp.int32, (BP, NER_V), 1)
              ).astype(jnp.float32)
    pos_emb = jnp.dot(pos_oh, post_ref[...], preferred_element_type=jnp.float32)
    ner_emb = jnp.dot(ner_oh, nert_ref[...], preferred_element_type=jnp.float32)
    h = (jnp.dot(p, w_word_ref[...], preferred_element_type=jnp.float32)
         + jnp.dot(sim, w_sim_ref[...], preferred_element_type=jnp.float32)
         + jnp.dot(pos_emb, w_pos_ref[...], preferred_element_type=jnp.float32)
         + jnp.dot(ner_emb, w_ner_ref[...], preferred_element_type=jnp.float32)
         + em_ref[...] * w_em_ref[...]
         + b1_ref[...])
    h = jnp.maximum(h, 0.0)
    pout_ref[...] = jnp.maximum(
        jnp.dot(h, w2_ref[...], preferred_element_type=jnp.float32)
        + b2_ref[...], 0.0)


def _tc_compute(p_emb, q_emb, pos2, ner2, exact_match, pos_table, ner_table,
                g_w, g_b, q_w1, q_b1, q_w2, q_b2,
                w_word, w_pos, w_ner, w_em, w_sim, b1, w2, b2,
                interpret=False):
    full = lambda shape: pl.BlockSpec(shape, lambda i: (0, 0))
    blk = lambda shape: pl.BlockSpec(shape, lambda i: (i, 0))
    return pl.pallas_call(
        _tc_body,
        grid=(P // BP,),
        in_specs=[
            blk((BP, D)),
            full((Q, D)),
            blk((BP, 1)),
            blk((BP, 1)),
            blk((BP, 1)),
            full((POS_V, POS_D)),
            full((NER_V, NER_D)),
            full((D, SIM_D)),
            full((1, SIM_D)),
            full((D, H1)),
            full((1, H1)),
            full((H1, H2)),
            full((1, H2)),
            full((D, H1)),
            full((POS_D, H1)),
            full((NER_D, H1)),
            full((1, H1)),
            full((SIM_D, H1)),
            full((1, H1)),
            full((H1, H2)),
            full((1, H2)),
        ],
        out_specs=[
            blk((BP, H2)),
            full((Q, H2)),
        ],
        out_shape=[
            jax.ShapeDtypeStruct((P, H2), jnp.float32),
            jax.ShapeDtypeStruct((Q, H2), jnp.float32),
        ],
        interpret=interpret,
    )(p_emb, q_emb, pos2, ner2, exact_match, pos_table, ner_table,
      g_w, g_b, q_w1, q_b1, q_w2, q_b2,
      w_word, w_pos, w_ner, w_em, w_sim, b1, w2, b2)


def kernel(paragraph_ids, paragraph_pos, paragraph_ner, exact_match,
           question_ids, word_table, pos_table, ner_table, g_w, g_b,
           q_w1, q_b1, q_w2, q_b2, d_w1, d_b1, d_w2, d_b2):
    table_wide = _transpose_pad(word_table.T)
    p_wide, q_wide = _sc_gather_wide(paragraph_ids, question_ids, table_wide)
    p_emb = p_wide[:, :D]
    q_emb = q_wide[:, :D]

    pos2 = paragraph_pos.reshape(P, 1)
    ner2 = paragraph_ner.reshape(P, 1)
    w_word = d_w1[0:D]
    w_pos = d_w1[D:D + POS_D]
    w_ner = d_w1[D + POS_D:D + POS_D + NER_D]
    w_em = d_w1[D + POS_D + NER_D:D + POS_D + NER_D + 1]
    w_sim = d_w1[D + POS_D + NER_D + 1:]

    pout, qout = _tc_compute(
        p_emb, q_emb, pos2, ner2, exact_match, pos_table, ner_table,
        g_w, g_b.reshape(1, SIM_D), q_w1, q_b1.reshape(1, H1),
        q_w2, q_b2.reshape(1, H2),
        w_word, w_pos, w_ner, w_em, w_sim,
        d_b1.reshape(1, H1), d_w2, d_b2.reshape(1, H2))

    return (pout[None], qout[None], p_emb, q_emb)


# speedup vs baseline: 4.4934x; 1.0600x over previous
import functools

import jax
import jax.numpy as jnp
from jax import lax
from jax.experimental import pallas as pl
from jax.experimental.pallas import tpu as pltpu
from jax.experimental.pallas import tpu_sc as plsc

P, Q, V, D = 4096, 128, 100000, 300
POS_V, POS_D, NER_V, NER_D = 50, 12, 20, 8
SIM_D, H1, H2 = 280, 512, 256

NC, NS = 2, 16
NW = NC * NS
PB_SC = P // NW
QB_SC = 8

BP = 1024


D_MAIN = 256
D_TAIL = D - D_MAIN
D_TAILP = 128

BLKE = 4096


def _transpose_pad(word_table_t):
    def body(in_ref, out_ref):
        out_ref[:, :D] = lax.transpose(in_ref[...], (1, 0))

    return pl.pallas_call(
        body,
        grid=(pl.cdiv(V, BLKE),),
        in_specs=[pl.BlockSpec((D, BLKE), lambda i: (0, i))],
        out_specs=pl.BlockSpec((BLKE, DWIDE), lambda i: (i, 0)),
        out_shape=jax.ShapeDtypeStruct((V, DWIDE), jnp.float32),
    )(word_table_t)


DWIDE = 384


def _sc_gather_wide(paragraph_ids, question_ids, table_wide):
    mesh = plsc.VectorSubcoreMesh(core_axis_name="c", subcore_axis_name="s")

    @functools.partial(
        pl.kernel,
        mesh=mesh,
        compiler_params=pltpu.CompilerParams(use_tc_tiling_on_sc=True),
        out_type=[
            jax.ShapeDtypeStruct((P, DWIDE), jnp.float32),
            jax.ShapeDtypeStruct((Q, DWIDE), jnp.float32),
        ],
        scratch_types=[
            pltpu.VMEM((PB_SC,), jnp.int32),
            pltpu.VMEM((PB_SC, DWIDE), jnp.float32),
            pltpu.VMEM((QB_SC,), jnp.int32),
            pltpu.VMEM((QB_SC, DWIDE), jnp.float32),
            pltpu.SemaphoreType.DMA,
            pltpu.SemaphoreType.DMA,
        ],
    )
    def gather_kernel(pids_hbm, qids_hbm, table_hbm, p_out, q_out,
                      pidx_v, prow_v, qidx_v, qrow_v, psem, qsem):
        wid = lax.axis_index("s") * NC + lax.axis_index("c")
        base = wid * PB_SC
        pltpu.sync_copy(pids_hbm.at[pl.ds(base, PB_SC)], pidx_v)
        pcopy = pltpu.async_copy(table_hbm.at[pidx_v], prow_v, psem)

        @pl.when(wid < Q // QB_SC)
        def _():
            qbase = wid * QB_SC
            pltpu.sync_copy(qids_hbm.at[pl.ds(qbase, QB_SC)], qidx_v)
            pltpu.async_copy(table_hbm.at[qidx_v], qrow_v, qsem).wait()
            pltpu.sync_copy(qrow_v, q_out.at[pl.ds(qbase, QB_SC)])

        pcopy.wait()
        pltpu.sync_copy(prow_v, p_out.at[pl.ds(base, PB_SC)])

    return gather_kernel(paragraph_ids, question_ids, table_wide)


def _tc_body(p_ref, q_ref, pos_ref, ner_ref, em_ref, post_ref, nert_ref,
             gw_ref, gb_ref, qw1_ref, qb1_ref, qw2_ref, qb2_ref,
             w_word_ref, w_pos_ref, w_ner_ref, w_em_ref, w_sim_ref, b1_ref,
             w2_ref, b2_ref, pout_ref, qout_ref):
    i = pl.program_id(0)
    q = q_ref[...]

    @pl.when(i == 0)
    def _():
        hq = jnp.maximum(
            jnp.dot(q, qw1_ref[...], preferred_element_type=jnp.float32)
            + qb1_ref[...], 0.0)
        qout_ref[...] = jnp.maximum(
            jnp.dot(hq, qw2_ref[...], preferred_element_type=jnp.float32)
            + qb2_ref[...], 0.0)

    p = p_ref[...]
    qn = q / (jnp.sqrt(jnp.sum(q * q, axis=1, keepdims=True)) + 1e-8)
    s = jnp.sum(qn, axis=0, keepdims=True)
    pnorm = jnp.sqrt(jnp.sum(p * p, axis=1, keepdims=True))
    ws = jnp.sum(p * s, axis=1, keepdims=True) / (pnorm + 1e-8)
    sim = jnp.maximum(
        jnp.dot(p, gw_ref[...], preferred_element_type=jnp.float32)
        + gb_ref[...], 0.0) * ws
    pos_oh = (pos_ref[...] == lax.broadcasted_iota(jnp.int32, (BP, POS_V), 1)
              ).astype(jnp.float32)
    ner_oh = (ner_ref[...] == lax.broadcasted_iota(jnp.int32, (BP, NER_V), 1)
              ).astype(jnp.float32)
    pos_emb = jnp.dot(pos_oh, post_ref[...], preferred_element_type=jnp.float32)
    ner_emb = jnp.dot(ner_oh, nert_ref[...], preferred_element_type=jnp.float32)
    h = (jnp.dot(p, w_word_ref[...], preferred_element_type=jnp.float32)
         + jnp.dot(sim, w_sim_ref[...], preferred_element_type=jnp.float32)
         + jnp.dot(pos_emb, w_pos_ref[...], preferred_element_type=jnp.float32)
         + jnp.dot(ner_emb, w_ner_ref[...], preferred_element_type=jnp.float32)
         + em_ref[...] * w_em_ref[...]
         + b1_ref[...])
    h = jnp.maximum(h, 0.0)
    pout_ref[...] = jnp.maximum(
        jnp.dot(h, w2_ref[...], preferred_element_type=jnp.float32)
        + b2_ref[...], 0.0)


def _tc_compute(p_emb, q_emb, pos2, ner2, exact_match, pos_table, ner_table,
                g_w, g_b, q_w1, q_b1, q_w2, q_b2,
                w_word, w_pos, w_ner, w_em, w_sim, b1, w2, b2,
                interpret=False):
    full = lambda shape: pl.BlockSpec(shape, lambda i: (0, 0))
    blk = lambda shape: pl.BlockSpec(shape, lambda i: (i, 0))
    return pl.pallas_call(
        _tc_body,
        grid=(P // BP,),
        in_specs=[
            blk((BP, D)),
            full((Q, D)),
            blk((BP, 1)),
            blk((BP, 1)),
            blk((BP, 1)),
            full((POS_V, POS_D)),
            full((NER_V, NER_D)),
            full((D, SIM_D)),
            full((1, SIM_D)),
            full((D, H1)),
            full((1, H1)),
            full((H1, H2)),
            full((1, H2)),
            full((D, H1)),
            full((POS_D, H1)),
            full((NER_D, H1)),
            full((1, H1)),
            full((SIM_D, H1)),
            full((1, H1)),
            full((H1, H2)),
            full((1, H2)),
        ],
        out_specs=[
            blk((BP, H2)),
            full((Q, H2)),
        ],
        out_shape=[
            jax.ShapeDtypeStruct((P, H2), jnp.float32),
            jax.ShapeDtypeStruct((Q, H2), jnp.float32),
        ],
        interpret=interpret,
    )(p_emb, q_emb, pos2, ner2, exact_match, pos_table, ner_table,
      g_w, g_b, q_w1, q_b1, q_w2, q_b2,
      w_word, w_pos, w_ner, w_em, w_sim, b1, w2, b2)


def kernel(paragraph_ids, paragraph_pos, paragraph_ner, exact_match,
           question_ids, word_table, pos_table, ner_table, g_w, g_b,
           q_w1, q_b1, q_w2, q_b2, d_w1, d_b1, d_w2, d_b2):
    table_wide = _transpose_pad(word_table.T)
    p_wide, q_wide = _sc_gather_wide(paragraph_ids, question_ids, table_wide)
    p_emb = p_wide[:, :D]
    q_emb = q_wide[:, :D]

    pos2 = paragraph_pos.reshape(P, 1)
    ner2 = paragraph_ner.reshape(P, 1)
    w_word = d_w1[0:D]
    w_pos = d_w1[D:D + POS_D]
    w_ner = d_w1[D + POS_D:D + POS_D + NER_D]
    w_em = d_w1[D + POS_D + NER_D:D + POS_D + NER_D + 1]
    w_sim = d_w1[D + POS_D + NER_D + 1:]

    pout, qout = _tc_compute(
        p_emb, q_emb, pos2, ner2, exact_match, pos_table, ner_table,
        g_w, g_b.reshape(1, SIM_D), q_w1, q_b1.reshape(1, H1),
        q_w2, q_b2.reshape(1, H2),
        w_word, w_pos, w_ner, w_em, w_sim,
        d_b1.reshape(1, H1), d_w2, d_b2.reshape(1, H2))

    return (pout[None], qout[None], p_emb, q_emb)


# speedup vs baseline: 4.5149x; 1.0048x over previous
import functools

import jax
import jax.numpy as jnp
from jax import lax
from jax.experimental import pallas as pl
from jax.experimental.pallas import tpu as pltpu
from jax.experimental.pallas import tpu_sc as plsc

P, Q, V, D = 4096, 128, 100000, 300
POS_V, POS_D, NER_V, NER_D = 50, 12, 20, 8
SIM_D, H1, H2 = 280, 512, 256

NC, NS = 2, 16
NW = NC * NS
PB_SC = P // NW
QB_SC = 8

BP = 2048


D_MAIN = 256
D_TAIL = D - D_MAIN
D_TAILP = 128

BLKE = 8192


def _transpose_pad(word_table_t):
    def body(in_ref, out_ref):
        out_ref[:, :D] = lax.transpose(in_ref[...], (1, 0))

    return pl.pallas_call(
        body,
        grid=(pl.cdiv(V, BLKE),),
        in_specs=[pl.BlockSpec((D, BLKE), lambda i: (0, i))],
        out_specs=pl.BlockSpec((BLKE, DWIDE), lambda i: (i, 0)),
        out_shape=jax.ShapeDtypeStruct((V, DWIDE), jnp.float32),
    )(word_table_t)


DWIDE = 384


def _sc_gather_wide(paragraph_ids, question_ids, table_wide):
    mesh = plsc.VectorSubcoreMesh(core_axis_name="c", subcore_axis_name="s")

    @functools.partial(
        pl.kernel,
        mesh=mesh,
        compiler_params=pltpu.CompilerParams(use_tc_tiling_on_sc=True),
        out_type=[
            jax.ShapeDtypeStruct((P, DWIDE), jnp.float32),
            jax.ShapeDtypeStruct((Q, DWIDE), jnp.float32),
        ],
        scratch_types=[
            pltpu.VMEM((PB_SC,), jnp.int32),
            pltpu.VMEM((PB_SC, DWIDE), jnp.float32),
            pltpu.VMEM((QB_SC,), jnp.int32),
            pltpu.VMEM((QB_SC, DWIDE), jnp.float32),
            pltpu.SemaphoreType.DMA,
            pltpu.SemaphoreType.DMA,
        ],
    )
    def gather_kernel(pids_hbm, qids_hbm, table_hbm, p_out, q_out,
                      pidx_v, prow_v, qidx_v, qrow_v, psem, qsem):
        wid = lax.axis_index("s") * NC + lax.axis_index("c")
        base = wid * PB_SC
        pltpu.sync_copy(pids_hbm.at[pl.ds(base, PB_SC)], pidx_v)
        pcopy = pltpu.async_copy(table_hbm.at[pidx_v], prow_v, psem)

        @pl.when(wid < Q // QB_SC)
        def _():
            qbase = wid * QB_SC
            pltpu.sync_copy(qids_hbm.at[pl.ds(qbase, QB_SC)], qidx_v)
            pltpu.async_copy(table_hbm.at[qidx_v], qrow_v, qsem).wait()
            pltpu.sync_copy(qrow_v, q_out.at[pl.ds(qbase, QB_SC)])

        pcopy.wait()
        pltpu.sync_copy(prow_v, p_out.at[pl.ds(base, PB_SC)])

    return gather_kernel(paragraph_ids, question_ids, table_wide)


def _tc_body(p_ref, q_ref, pos_ref, ner_ref, em_ref, post_ref, nert_ref,
             gw_ref, gb_ref, qw1_ref, qb1_ref, qw2_ref, qb2_ref,
             w_word_ref, w_pos_ref, w_ner_ref, w_em_ref, w_sim_ref, b1_ref,
             w2_ref, b2_ref, pout_ref, qout_ref):
    i = pl.program_id(0)
    q = q_ref[...]

    @pl.when(i == 0)
    def _():
        hq = jnp.maximum(
            jnp.dot(q, qw1_ref[...], preferred_element_type=jnp.float32)
            + qb1_ref[...], 0.0)
        qout_ref[...] = jnp.maximum(
            jnp.dot(hq, qw2_ref[...], preferred_element_type=jnp.float32)
            + qb2_ref[...], 0.0)

    p = p_ref[...]
    qn = q / (jnp.sqrt(jnp.sum(q * q, axis=1, keepdims=True)) + 1e-8)
    s = jnp.sum(qn, axis=0, keepdims=True)
    pnorm = jnp.sqrt(jnp.sum(p * p, axis=1, keepdims=True))
    ws = jnp.sum(p * s, axis=1, keepdims=True) / (pnorm + 1e-8)
    sim = jnp.maximum(
        jnp.dot(p, gw_ref[...], preferred_element_type=jnp.float32)
        + gb_ref[...], 0.0) * ws
    pos_oh = (pos_ref[...] == lax.broadcasted_iota(jnp.int32, (BP, POS_V), 1)
              ).astype(jnp.float32)
    ner_oh = (ner_ref[...] == lax.broadcasted_iota(jnp.int32, (BP, NER_V), 1)
              ).astype(jnp.float32)
    pos_emb = jnp.dot(pos_oh, post_ref[...], preferred_element_type=jnp.float32)
    ner_emb = jnp.dot(ner_oh, nert_ref[...], preferred_element_type=jnp.float32)
    h = (jnp.dot(p, w_word_ref[...], preferred_element_type=jnp.float32)
         + jnp.dot(sim, w_sim_ref[...], preferred_element_type=jnp.float32)
         + jnp.dot(pos_emb, w_pos_ref[...], preferred_element_type=jnp.float32)
         + jnp.dot(ner_emb, w_ner_ref[...], preferred_element_type=jnp.float32)
         + em_ref[...] * w_em_ref[...]
         + b1_ref[...])
    h = jnp.maximum(h, 0.0)
    pout_ref[...] = jnp.maximum(
        jnp.dot(h, w2_ref[...], preferred_element_type=jnp.float32)
        + b2_ref[...], 0.0)


def _tc_compute(p_emb, q_emb, pos2, ner2, exact_match, pos_table, ner_table,
                g_w, g_b, q_w1, q_b1, q_w2, q_b2,
                w_word, w_pos, w_ner, w_em, w_sim, b1, w2, b2,
                interpret=False):
    full = lambda shape: pl.BlockSpec(shape, lambda i: (0, 0))
    blk = lambda shape: pl.BlockSpec(shape, lambda i: (i, 0))
    return pl.pallas_call(
        _tc_body,
        grid=(P // BP,),
        in_specs=[
            blk((BP, D)),
            full((Q, D)),
            blk((BP, 1)),
            blk((BP, 1)),
            blk((BP, 1)),
            full((POS_V, POS_D)),
            full((NER_V, NER_D)),
            full((D, SIM_D)),
            full((1, SIM_D)),
            full((D, H1)),
            full((1, H1)),
            full((H1, H2)),
            full((1, H2)),
            full((D, H1)),
            full((POS_D, H1)),
            full((NER_D, H1)),
            full((1, H1)),
            full((SIM_D, H1)),
            full((1, H1)),
            full((H1, H2)),
            full((1, H2)),
        ],
        out_specs=[
            blk((BP, H2)),
            full((Q, H2)),
        ],
        out_shape=[
            jax.ShapeDtypeStruct((P, H2), jnp.float32),
            jax.ShapeDtypeStruct((Q, H2), jnp.float32),
        ],
        interpret=interpret,
    )(p_emb, q_emb, pos2, ner2, exact_match, pos_table, ner_table,
      g_w, g_b, q_w1, q_b1, q_w2, q_b2,
      w_word, w_pos, w_ner, w_em, w_sim, b1, w2, b2)


def kernel(paragraph_ids, paragraph_pos, paragraph_ner, exact_match,
           question_ids, word_table, pos_table, ner_table, g_w, g_b,
           q_w1, q_b1, q_w2, q_b2, d_w1, d_b1, d_w2, d_b2):
    table_wide = _transpose_pad(word_table.T)
    p_wide, q_wide = _sc_gather_wide(paragraph_ids, question_ids, table_wide)
    p_emb = p_wide[:, :D]
    q_emb = q_wide[:, :D]

    pos2 = paragraph_pos.reshape(P, 1)
    ner2 = paragraph_ner.reshape(P, 1)
    w_word = d_w1[0:D]
    w_pos = d_w1[D:D + POS_D]
    w_ner = d_w1[D + POS_D:D + POS_D + NER_D]
    w_em = d_w1[D + POS_D + NER_D:D + POS_D + NER_D + 1]
    w_sim = d_w1[D + POS_D + NER_D + 1:]

    pout, qout = _tc_compute(
        p_emb, q_emb, pos2, ner2, exact_match, pos_table, ner_table,
        g_w, g_b.reshape(1, SIM_D), q_w1, q_b1.reshape(1, H1),
        q_w2, q_b2.reshape(1, H2),
        w_word, w_pos, w_ner, w_em, w_sim,
        d_b1.reshape(1, H1), d_w2, d_b2.reshape(1, H2))

    return (pout[None], qout[None], p_emb, q_emb)
